# baseline (speedup 1.0000x reference)
import functools

import jax
import jax.numpy as jnp
from jax import lax
from jax.experimental import pallas as pl
from jax.experimental.pallas import tpu as pltpu
from jax.experimental.pallas import tpu_sc as plsc

L = 16
NC, NS = 2, 16
NW = NC * NS

N_ROWS = 8192
D = 2048
N_CLS_PAD = 1024
N_TGT_PAD = 112
N_TGT = 100

ROWS_PER_W = N_ROWS // NW
CHUNK = 8
N_CHUNKS = ROWS_PER_W // CHUNK
KV = D // L


def _body(feat_hbm, lab_hbm, tgt_hbm, pts_hbm, out_hbm,
          lab_v, tgt_v, counts_v, m_v, feat_v, pts_v, out_v, sem):
    c = lax.axis_index("c")
    s = lax.axis_index("s")
    wid = c * NS + s
    base = wid * ROWS_PER_W

    pltpu.sync_copy(lab_hbm.at[pl.ds(base, ROWS_PER_W)], lab_v)
    pltpu.sync_copy(tgt_hbm, tgt_v)

    zeros_i = jnp.zeros((L,), jnp.int32)
    for i in range(N_CLS_PAD // L):
        counts_v[pl.ds(i * L, L)] = zeros_i
    ones_i = jnp.ones((L,), jnp.int32)
    lane = lax.iota(jnp.int32, L)
    for i in range(N_TGT_PAD // L):
        t = tgt_v[pl.ds(i * L, L)]
        msk = (lane + (i * L)) < N_TGT
        plsc.addupdate_scatter(counts_v, [t], ones_i, mask=msk)

    for i in range(ROWS_PER_W // L):
        lab = lab_v[pl.ds(i * L, L)]
        cnt = plsc.load_gather(counts_v, [lab])
        m_v[pl.ds(i * L, L)] = cnt.astype(jnp.float32)

    def chunk_body(ci, total):
        row0 = base + ci * CHUNK
        pltpu.sync_copy(feat_hbm.at[pl.ds(row0, CHUNK)], feat_v)
        idx = lab_v.at[pl.ds(ci * CHUNK, CHUNK)]
        pltpu.async_copy(pts_hbm.at[idx], pts_v, sem).wait()

        def row_body(r, tot):
            accs = [jnp.zeros((L,), jnp.float32) for _ in range(4)]
            for k in range(KV):
                f = feat_v[r, pl.ds(k * L, L)]
                p = pts_v[r, pl.ds(k * L, L)]
                accs[k % 4] = accs[k % 4] + f * p
            acc = (accs[0] + accs[1]) + (accs[2] + accs[3])
            d = jnp.sum(acc)
            ridx = jnp.full((L,), ci * CHUNK + r, jnp.int32)
            m = plsc.load_gather(m_v, [ridx])[0]
            e = m * d - 1.0
            return tot + 0.5 * e * e

        return lax.fori_loop(0, CHUNK, row_body, total)

    total = lax.fori_loop(0, N_CHUNKS, chunk_body, jnp.float32(0.0))

    out_v[...] = jnp.full((L,), total, jnp.float32)
    pltpu.sync_copy(out_v, out_hbm.at[wid])


@jax.jit
def _sc_loss(feat2, rep_labels, tgt_pad, points):
    mesh = plsc.VectorSubcoreMesh(core_axis_name="c", subcore_axis_name="s")
    run = functools.partial(
        pl.kernel,
        out_type=jax.ShapeDtypeStruct((NW, L), jnp.float32),
        mesh=mesh,
        scratch_types=[
            pltpu.VMEM((ROWS_PER_W,), jnp.int32),
            pltpu.VMEM((N_TGT_PAD,), jnp.int32),
            pltpu.VMEM((N_CLS_PAD,), jnp.int32),
            pltpu.VMEM((ROWS_PER_W,), jnp.float32),
            pltpu.VMEM((CHUNK, D), jnp.float32),
            pltpu.VMEM((CHUNK, D), jnp.float32),
            pltpu.VMEM((L,), jnp.float32),
            pltpu.SemaphoreType.DMA,
        ],
        compiler_params=pltpu.CompilerParams(needs_layout_passes=False),
    )(_body)
    partials = run(feat2, rep_labels, tgt_pad, points)
    return jnp.sum(partials[:, 0]) / jnp.float32(N_ROWS)


def kernel(features, labels, target_labels, points):
    bsz, n_views, d = features.shape
    feat2 = features.reshape(bsz * n_views, d)
    rep_labels = jnp.repeat(labels, n_views)
    tgt_pad = jnp.concatenate(
        [target_labels, jnp.zeros((N_TGT_PAD - N_TGT,), jnp.int32)])
    return _sc_loss(feat2, rep_labels, tgt_pad, points)


# speedup vs baseline: 2.1908x; 2.1908x over previous
import functools

import jax
import jax.numpy as jnp
from jax import lax
from jax.experimental import pallas as pl
from jax.experimental.pallas import tpu as pltpu
from jax.experimental.pallas import tpu_sc as plsc

L = 16
NC, NS = 2, 16
NW = NC * NS

N_ROWS = 8192
D = 2048
N_CLS_PAD = 1024
N_TGT_PAD = 112
N_TGT = 100

ROWS_PER_W = N_ROWS // NW
CHUNK = 8
N_CHUNKS = ROWS_PER_W // CHUNK
KV = D // L


LIVE_PAD = ROWS_PER_W + L


def _body(feat_hbm, lab_hbm, tgt_hbm, pts_hbm, out_hbm,
          lab_v, tgt_v, counts_v, live_idx_v, live_lab_v, live_m_v,
          feat_v, pts_v, out_v, sem):
    c = lax.axis_index("c")
    s = lax.axis_index("s")
    wid = c * NS + s
    base = wid * ROWS_PER_W

    pltpu.sync_copy(lab_hbm.at[pl.ds(base, ROWS_PER_W)], lab_v)
    pltpu.sync_copy(tgt_hbm, tgt_v)

    zeros_i = jnp.zeros((L,), jnp.int32)
    zeros_f = jnp.zeros((L,), jnp.float32)
    for i in range(N_CLS_PAD // L):
        counts_v[pl.ds(i * L, L)] = zeros_i
    for i in range(LIVE_PAD // L):
        live_idx_v[pl.ds(i * L, L)] = zeros_i
        live_lab_v[pl.ds(i * L, L)] = zeros_i
        live_m_v[pl.ds(i * L, L)] = zeros_f
    ones_i = jnp.ones((L,), jnp.int32)
    lane = lax.iota(jnp.int32, L)
    for i in range(N_TGT_PAD // L):
        t = tgt_v[pl.ds(i * L, L)]
        msk = (lane + (i * L)) < N_TGT
        plsc.addupdate_scatter(counts_v, [t], ones_i, mask=msk)

    def comp_body(i, n_live):
        lab = lab_v[pl.ds(i * L, L)]
        cnt = plsc.load_gather(counts_v, [lab])
        msk = cnt != 0
        pos = n_live + plsc.cumsum(msk.astype(jnp.int32)) - 1
        glob = (base + i * L) + lane
        plsc.store_scatter(live_idx_v, [pos], glob, mask=msk)
        plsc.store_scatter(live_lab_v, [pos], lab, mask=msk)
        plsc.store_scatter(live_m_v, [pos], cnt.astype(jnp.float32), mask=msk)
        return n_live + plsc.all_reduce_population_count(msk)[0]

    n_live = lax.fori_loop(0, ROWS_PER_W // L, comp_body, jnp.int32(0))
    n_chunks = (n_live + (CHUNK - 1)) // CHUNK

    def chunk_body(ci, total):
        fidx = live_idx_v.at[pl.ds(ci * CHUNK, CHUNK)]
        pidx = live_lab_v.at[pl.ds(ci * CHUNK, CHUNK)]
        cp_f = pltpu.async_copy(feat_hbm.at[fidx], feat_v, sem)
        cp_p = pltpu.async_copy(pts_hbm.at[pidx], pts_v, sem)
        cp_f.wait()
        cp_p.wait()

        def row_body(r, tot):
            accs = [jnp.zeros((L,), jnp.float32) for _ in range(4)]
            for k in range(KV):
                f = feat_v[r, pl.ds(k * L, L)]
                p = pts_v[r, pl.ds(k * L, L)]
                accs[k % 4] = accs[k % 4] + f * p
            acc = (accs[0] + accs[1]) + (accs[2] + accs[3])
            d = jnp.sum(acc)
            rw = ci * CHUNK + r
            m = plsc.load_gather(live_m_v, [jnp.full((L,), rw, jnp.int32)])[0]
            e = m * d - 1.0
            return tot + jnp.where(rw < n_live, 0.5 * e * e, 0.0)

        return lax.fori_loop(0, CHUNK, row_body, total)

    total = lax.fori_loop(0, n_chunks, chunk_body, jnp.float32(0.0))
    total = total + 0.5 * (ROWS_PER_W - n_live).astype(jnp.float32)

    out_v[...] = jnp.full((L,), total, jnp.float32)
    pltpu.sync_copy(out_v, out_hbm.at[wid])


@jax.jit
def _sc_loss(feat2, rep_labels, tgt_pad, points):
    mesh = plsc.VectorSubcoreMesh(core_axis_name="c", subcore_axis_name="s")
    run = functools.partial(
        pl.kernel,
        out_type=jax.ShapeDtypeStruct((NW, L), jnp.float32),
        mesh=mesh,
        scratch_types=[
            pltpu.VMEM((ROWS_PER_W,), jnp.int32),
            pltpu.VMEM((N_TGT_PAD,), jnp.int32),
            pltpu.VMEM((N_CLS_PAD,), jnp.int32),
            pltpu.VMEM((LIVE_PAD,), jnp.int32),
            pltpu.VMEM((LIVE_PAD,), jnp.int32),
            pltpu.VMEM((LIVE_PAD,), jnp.float32),
            pltpu.VMEM((CHUNK, D), jnp.float32),
            pltpu.VMEM((CHUNK, D), jnp.float32),
            pltpu.VMEM((L,), jnp.float32),
            pltpu.SemaphoreType.DMA,
        ],
        compiler_params=pltpu.CompilerParams(needs_layout_passes=False),
    )(_body)
    partials = run(feat2, rep_labels, tgt_pad, points)
    return jnp.sum(partials[:, 0]) / jnp.float32(N_ROWS)


def kernel(features, labels, target_labels, points):
    bsz, n_views, d = features.shape
    feat2 = features.reshape(bsz * n_views, d)
    rep_labels = jnp.repeat(labels, n_views)
    tgt_pad = jnp.concatenate(
        [target_labels, jnp.zeros((N_TGT_PAD - N_TGT,), jnp.int32)])
    return _sc_loss(feat2, rep_labels, tgt_pad, points)
